# Initial kernel scaffold; baseline (speedup 1.0000x reference)
import jax
import jax.numpy as jnp
from jax.experimental import pallas as pl


def kernel(x, edge_index, edge_attr, batch, W_self, W_nbr, W_edge, b_enc, We1, be1, We2, be2, Wc1, bc1, Wc2, bc2, Wv1, bv1, Wv2, bv2, Whc, bhc, Whe, bhe):
    raise NotImplementedError("write your pallas kernel here")


# speedup vs baseline: 2.0680x; 2.0680x over previous
import functools

import jax
import jax.numpy as jnp
from jax import lax
from jax.experimental import pallas as pl
from jax.experimental.pallas import tpu as pltpu
from jax.experimental.pallas import tpu_sc as plsc

N = 10000
E = 320000
D = 128
DE = 16
B = 512
BOT = 256
ND = 5

NC = 2
NS = 16

CH = 128
NCHUNKS = E // CH
ITERS = (NCHUNKS + NS - 1) // NS

NH = N // NC
NHP = 5040
ZR = 80
NQZ = NHP // ZR
QIZ = (NQZ + NS - 1) // NS
WR = 40
NQW = NH // WR
QIW = (NQW + NS - 1) // NS


def _sc_edge_agg_body(src_hbm, dst_hbm, x_hbm, ea_hbm, out_ax, out_ae,
                      src_v, dst_v, dloc_v, rows_v, ea_v, zrow, bnc,
                      acc_sh, sem):
    cid = lax.axis_index("c")
    sid = lax.axis_index("s")
    lo = cid * NH

    def zero_zrow(i, _):
        for j in range(D // 16):
            zrow[i, pl.ds(j * 16, 16)] = jnp.zeros((16,), jnp.float32)
        return 0
    lax.fori_loop(0, ZR, zero_zrow, 0)

    def zero_rows_v(i, _):
        for j in range(D // 16):
            rows_v[i, pl.ds(j * 16, 16)] = jnp.zeros((16,), jnp.float32)
        return 0

    def zero_acc(k, _):
        q = sid + k * NS

        @pl.when(q < NQZ)
        def _():
            pltpu.sync_copy(zrow, acc_sh.at[pl.ds(q * ZR, ZR)])
        return 0

    def compute_dloc():
        for j in range(CH // 16):
            d = dst_v[pl.ds(j * 16, 16)]
            m = (d >= lo) & (d < lo + NH)
            dloc_v[pl.ds(j * 16, 16)] = jnp.where(m, d - lo, NH)

    def writeout(out_hbm):
        def wo(k, _):
            q = sid + k * NS

            @pl.when(q < NQW)
            def _():
                r = q * WR
                pltpu.sync_copy(acc_sh.at[pl.ds(r, WR)], bnc)
                pltpu.sync_copy(bnc, out_hbm.at[pl.ds(lo + r, WR)])
            return 0
        lax.fori_loop(0, QIW, wo, 0)

    lax.fori_loop(0, QIZ, zero_acc, 0)
    plsc.subcore_barrier()

    def body1(i, _):
        c = sid + i * NS

        @pl.when(c < NCHUNKS)
        def _():
            base = c * CH
            pltpu.sync_copy(src_hbm.at[pl.ds(base, CH)], src_v)
            pltpu.sync_copy(dst_hbm.at[pl.ds(base, CH)], dst_v)
            pltpu.async_copy(x_hbm.at[src_v], rows_v, sem).wait()
            compute_dloc()
            pltpu.sync_copy(rows_v, acc_sh.at[dloc_v], add=True)
        return 0
    lax.fori_loop(0, ITERS, body1, 0)

    plsc.subcore_barrier()
    writeout(out_ax)
    plsc.subcore_barrier()

    lax.fori_loop(0, QIZ, zero_acc, 0)
    lax.fori_loop(0, CH, zero_rows_v, 0)
    plsc.subcore_barrier()

    def body2(i, _):
        c = sid + i * NS

        @pl.when(c < NCHUNKS)
        def _():
            base = c * CH
            pltpu.sync_copy(dst_hbm.at[pl.ds(base, CH)], dst_v)
            pltpu.sync_copy(ea_hbm.at[pl.ds(base, CH)], ea_v)

            def cp(r, _):
                rows_v[r, pl.ds(0, DE)] = ea_v[r, pl.ds(0, DE)]
                return 0
            lax.fori_loop(0, CH, cp, 0)
            compute_dloc()
            pltpu.sync_copy(rows_v, acc_sh.at[dloc_v], add=True)
        return 0
    lax.fori_loop(0, ITERS, body2, 0)

    plsc.subcore_barrier()
    writeout(out_ae)


@jax.jit
def _sc_edge_agg(src, dst, x, ea):
    mesh = plsc.VectorSubcoreMesh(core_axis_name="c", subcore_axis_name="s")
    fn = pl.kernel(
        _sc_edge_agg_body,
        out_type=[
            jax.ShapeDtypeStruct((N, D), jnp.float32),
            jax.ShapeDtypeStruct((N, D), jnp.float32),
        ],
        mesh=mesh,
        scratch_types=[
            pltpu.VMEM((CH,), jnp.int32),
            pltpu.VMEM((CH,), jnp.int32),
            pltpu.VMEM((CH,), jnp.int32),
            pltpu.VMEM((CH, D), jnp.float32),
            pltpu.VMEM((CH, DE), jnp.float32),
            pltpu.VMEM((ZR, D), jnp.float32),
            pltpu.VMEM((WR, D), jnp.float32),
            pltpu.VMEM_SHARED((NHP, D), jnp.float32),
            pltpu.SemaphoreType.DMA,
        ],
    )
    return fn(src, dst, x, ea)


TN = 1000
GRID = N // TN


def _tc_dense_body(x_ref, ax_ref, ae_ref, bf_ref,
                   Wself_ref, Wnbr_ref, Wedge_ref, benc_ref,
                   We1_ref, be1_ref, We2_ref, be2_ref,
                   Wc1_ref, bc1_ref, Wc2_ref, bc2_ref,
                   Wv1_ref, bv1_ref, Wv2_ref, bv2_ref,
                   Whc_ref, bhc_ref, Whe_ref, bhe_ref,
                   mask_ref, predc_ref, prede_ref,
                   Gc, Gh, Cn):
    i = pl.program_id(0)

    @pl.when(i == 0)
    def _():
        Gc[...] = jnp.zeros_like(Gc)
        Gh[...] = jnp.zeros_like(Gh)
        Cn[...] = jnp.zeros_like(Cn)

    h = (x_ref[...] @ Wself_ref[...] + ax_ref[...] @ Wnbr_ref[...]
         + ae_ref[...] @ Wedge_ref[...])
    h = jnp.maximum(h + benc_ref[...], 0.0)

    m1 = jnp.maximum(h @ We1_ref[...] + be1_ref[...], 0.0)
    logits = m1 @ We2_ref[...] + be2_ref[0, 0]
    mask = 1.0 / (1.0 + jnp.exp(-logits))
    mask_ref[...] = mask
    hc = h * mask

    bf = bf_ref[...]
    cols = lax.broadcasted_iota(jnp.int32, (TN, B), 1).astype(jnp.float32)
    P = jnp.where(bf == cols, 1.0, 0.0)

    dims = (((0,), (0,)), ((), ()))
    Gc[...] += lax.dot_general(P, hc, dims, preferred_element_type=jnp.float32)
    Gh[...] += lax.dot_general(P, h, dims, preferred_element_type=jnp.float32)
    Cn[...] += lax.dot_general(P, jnp.ones((TN, D), jnp.float32), dims,
                               preferred_element_type=jnp.float32)

    @pl.when(i == GRID - 1)
    def _():
        cnt = jnp.maximum(Cn[...], 1.0)
        hgc = Gc[...] / cnt
        hge = (Gh[...] - Gc[...]) / cnt
        t = jnp.maximum(hgc @ Wc1_ref[...] + bc1_ref[...], 0.0)
        hsc = jnp.maximum(t @ Wc2_ref[...] + bc2_ref[...], 0.0)
        t = jnp.maximum(hge @ Wv1_ref[...] + bv1_ref[...], 0.0)
        hse = jnp.maximum(t @ Wv2_ref[...] + bv2_ref[...], 0.0)
        predc_ref[...] = hsc @ Whc_ref[...] + bhc_ref[...]
        prede_ref[...] = hse @ Whe_ref[...] + bhe_ref[...]


def _full(shape):
    return pl.BlockSpec(shape, lambda i: (0,) * len(shape))


@jax.jit
def _tc_dense(x, ax, ae, batch_f,
              W_self, W_nbr, W_edge, b_enc,
              We1, be1, We2, be2,
              Wc1, bc1, Wc2, bc2,
              Wv1, bv1, Wv2, bv2,
              Whc_p, bhc_p, Whe_p, bhe_p):
    return pl.pallas_call(
        _tc_dense_body,
        grid=(GRID,),
        in_specs=[
            pl.BlockSpec((TN, D), lambda i: (i, 0)),
            pl.BlockSpec((TN, D), lambda i: (i, 0)),
            pl.BlockSpec((TN, D), lambda i: (i, 0)),
            pl.BlockSpec((TN, 1), lambda i: (i, 0)),
            _full((D, D)), _full((D, D)), _full((D, D)), _full((1, D)),
            _full((D, D // 2)), _full((1, D // 2)), _full((D // 2, 1)),
            _full((1, 1)),
            _full((D, D)), _full((1, D)), _full((D, BOT)), _full((1, BOT)),
            _full((D, D)), _full((1, D)), _full((D, BOT)), _full((1, BOT)),
            _full((BOT, D)), _full((1, D)), _full((BOT, D)), _full((1, D)),
        ],
        out_specs=[
            pl.BlockSpec((TN, 1), lambda i: (i, 0)),
            pl.BlockSpec((B, D), lambda i: (0, 0)),
            pl.BlockSpec((B, D), lambda i: (0, 0)),
        ],
        out_shape=[
            jax.ShapeDtypeStruct((N, 1), jnp.float32),
            jax.ShapeDtypeStruct((B, D), jnp.float32),
            jax.ShapeDtypeStruct((B, D), jnp.float32),
        ],
        scratch_shapes=[
            pltpu.VMEM((B, D), jnp.float32),
            pltpu.VMEM((B, D), jnp.float32),
            pltpu.VMEM((B, D), jnp.float32),
        ],
    )(x, ax, ae, batch_f,
      W_self, W_nbr, W_edge, b_enc,
      We1, be1, We2, be2,
      Wc1, bc1, Wc2, bc2,
      Wv1, bv1, Wv2, bv2,
      Whc_p, bhc_p, Whe_p, bhe_p)


def kernel(x, edge_index, edge_attr, batch,
           W_self, W_nbr, W_edge, b_enc,
           We1, be1, We2, be2,
           Wc1, bc1, Wc2, bc2,
           Wv1, bv1, Wv2, bv2,
           Whc, bhc, Whe, bhe):
    src = edge_index[0].astype(jnp.int32)
    dst = edge_index[1].astype(jnp.int32)

    ax, ae = _sc_edge_agg(src, dst, x, edge_attr)
    W_edge_p = jnp.pad(W_edge, ((0, D - DE), (0, 0)))

    batch_f = batch.astype(jnp.float32).reshape(N, 1)
    pad = lambda w: jnp.pad(w, ((0, 0), (0, D - w.shape[1])))
    Whc_p = pad(Whc)
    Whe_p = pad(Whe)
    bhc_p = jnp.pad(bhc, (0, D - ND)).reshape(1, D)
    bhe_p = jnp.pad(bhe, (0, D - ND)).reshape(1, D)

    mask, predc_full, prede_full = _tc_dense(
        x, ax, ae, batch_f,
        W_self, W_nbr, W_edge_p, b_enc.reshape(1, D),
        We1, be1.reshape(1, D // 2), We2, be2.reshape(1, 1),
        Wc1, bc1.reshape(1, D), Wc2, bc2.reshape(1, BOT),
        Wv1, bv1.reshape(1, D), Wv2, bv2.reshape(1, BOT),
        Whc_p, bhc_p, Whe_p, bhe_p)

    return (predc_full[:, :ND], prede_full[:, :ND], mask)


# speedup vs baseline: 3.4490x; 1.6678x over previous
import functools

import jax
import jax.numpy as jnp
from jax import lax
from jax.experimental import pallas as pl
from jax.experimental.pallas import tpu as pltpu
from jax.experimental.pallas import tpu_sc as plsc

N = 10000
E = 320000
D = 128
DE = 16
B = 512
BOT = 256
ND = 5

NC = 2
NS = 16

CH = 128
NCHUNKS = E // CH
NFULL = NCHUNKS // NS
PAIRS = NFULL // 2
NTAIL = NCHUNKS - NFULL * NS

NH = N // NC
NHP = 5040
ZR = 80
NQZ = NHP // ZR
QIZ = (NQZ + NS - 1) // NS
WR = 40
NQW = NH // WR
QIW = (NQW + NS - 1) // NS


def _sc_edge_agg_body(packed_hbm, x_hbm, ea_hbm, out_ax, out_ae,
                      src0, src1, dst0, dst1, dloc0, dloc1,
                      rows0, rows1, ea0, ea1, zrow, bnc, acc_sh,
                      sg0, sg1, ss0, ss1, si0, si1):
    cid = lax.axis_index("c")
    sid = lax.axis_index("s")
    lo = cid * NH

    SRC = (src0, src1)
    DST = (dst0, dst1)
    DLOC = (dloc0, dloc1)
    ROWS = (rows0, rows1)
    EA = (ea0, ea1)
    SG = (sg0, sg1)
    SS = (ss0, ss1)
    SI = (si0, si1)

    def zero_zrow(i, _):
        for j in range(D // 16):
            zrow[i, pl.ds(j * 16, 16)] = jnp.zeros((16,), jnp.float32)
        return 0
    lax.fori_loop(0, ZR, zero_zrow, 0)

    def zero_acc(k, _):
        q = sid + k * NS

        @pl.when(q < NQZ)
        def _():
            pltpu.sync_copy(zrow, acc_sh.at[pl.ds(q * ZR, ZR)])
        return 0

    def fill_dummy(dloc):
        for j in range(CH // 16):
            dloc[pl.ds(j * 16, 16)] = jnp.full((16,), NH, jnp.int32)

    def compute_dloc(dst, dloc):
        for j in range(CH // 16):
            d = dst[pl.ds(j * 16, 16)]
            m = (d >= lo) & (d < lo + NH)
            dloc[pl.ds(j * 16, 16)] = jnp.where(m, d - lo, NH)

    def writeout(out_hbm):
        def wo(k, _):
            q = sid + k * NS

            @pl.when(q < NQW)
            def _():
                r = q * WR
                pltpu.sync_copy(acc_sh.at[pl.ds(r, WR)], bnc)
                pltpu.sync_copy(bnc, out_hbm.at[pl.ds(lo + r, WR)])
            return 0
        lax.fori_loop(0, QIW, wo, 0)

    lax.fori_loop(0, QIZ, zero_acc, 0)
    fill_dummy(dloc0)
    fill_dummy(dloc1)
    plsc.subcore_barrier()

    pltpu.async_copy(rows0, acc_sh.at[dloc0], ss0, add=True)
    pltpu.async_copy(rows1, acc_sh.at[dloc1], ss1, add=True)
    pltpu.sync_copy(packed_hbm.at[sid, 0], src0)
    pltpu.sync_copy(packed_hbm.at[sid, 1], dst0)
    pltpu.async_copy(packed_hbm.at[sid + NS, 0], src1, si1)
    pltpu.async_copy(packed_hbm.at[sid + NS, 1], dst1, si1)
    pltpu.async_copy(x_hbm.at[src0], rows0, sg0)

    def pair1(k, _):
        i0 = 2 * k
        for b in (0, 1):
            nb = 1 - b
            ii = i0 + b
            pltpu.make_async_copy(x_hbm.at[SRC[b]], ROWS[b], SG[b]).wait()
            compute_dloc(DST[b], DLOC[b])
            pltpu.async_copy(ROWS[b], acc_sh.at[DLOC[b]], SS[b], add=True)

            @pl.when(ii + 2 < NFULL)
            def _():
                c2 = sid + (ii + 2) * NS
                pltpu.async_copy(packed_hbm.at[c2, 0], SRC[b], SI[b])
                pltpu.async_copy(packed_hbm.at[c2, 1], DST[b], SI[b])

            @pl.when(ii + 1 < NFULL)
            def _():
                pltpu.make_async_copy(packed_hbm.at[0, 0], SRC[nb], SI[nb]).wait()
                pltpu.make_async_copy(packed_hbm.at[0, 1], DST[nb], SI[nb]).wait()
                pltpu.make_async_copy(ROWS[nb], acc_sh.at[DLOC[nb]], SS[nb]).wait()
                pltpu.async_copy(x_hbm.at[SRC[nb]], ROWS[nb], SG[nb])
        return 0
    lax.fori_loop(0, PAIRS, pair1, 0)

    pltpu.make_async_copy(rows0, acc_sh.at[dloc0], ss0).wait()
    pltpu.make_async_copy(rows1, acc_sh.at[dloc1], ss1).wait()

    @pl.when(sid < NTAIL)
    def _():
        c = sid + NFULL * NS
        pltpu.sync_copy(packed_hbm.at[c, 0], src0)
        pltpu.sync_copy(packed_hbm.at[c, 1], dst0)
        pltpu.async_copy(x_hbm.at[src0], rows0, sg0).wait()
        compute_dloc(dst0, dloc0)
        pltpu.sync_copy(rows0, acc_sh.at[dloc0], add=True)

    plsc.subcore_barrier()
    writeout(out_ax)
    plsc.subcore_barrier()

    lax.fori_loop(0, QIZ, zero_acc, 0)

    def zero_rows(i, _):
        for j in range(D // 16):
            rows0[i, pl.ds(j * 16, 16)] = jnp.zeros((16,), jnp.float32)
            rows1[i, pl.ds(j * 16, 16)] = jnp.zeros((16,), jnp.float32)
        return 0
    lax.fori_loop(0, CH, zero_rows, 0)
    fill_dummy(dloc0)
    fill_dummy(dloc1)
    plsc.subcore_barrier()

    pltpu.async_copy(packed_hbm.at[sid, 1], dst0, si0)
    pltpu.async_copy(ea_hbm.at[pl.ds(sid * CH, CH)], ea0, si0)
    pltpu.async_copy(packed_hbm.at[sid + NS, 1], dst1, si1)
    pltpu.async_copy(ea_hbm.at[pl.ds((sid + NS) * CH, CH)], ea1, si1)

    def pair2(k, _):
        i0 = 2 * k
        for b in (0, 1):
            ii = i0 + b
            pltpu.make_async_copy(packed_hbm.at[0, 1], DST[b], SI[b]).wait()
            pltpu.make_async_copy(ea_hbm.at[pl.ds(0, CH)], EA[b], SI[b]).wait()

            def cp(r, _):
                ROWS[b][r, pl.ds(0, DE)] = EA[b][r, pl.ds(0, DE)]
                return 0
            lax.fori_loop(0, CH, cp, 0)
            compute_dloc(DST[b], DLOC[b])

            @pl.when(ii + 2 < NFULL)
            def _():
                c2 = sid + (ii + 2) * NS
                pltpu.async_copy(packed_hbm.at[c2, 1], DST[b], SI[b])
                pltpu.async_copy(ea_hbm.at[pl.ds(c2 * CH, CH)], EA[b], SI[b])
            pltpu.sync_copy(ROWS[b], acc_sh.at[DLOC[b]], add=True)
        return 0
    lax.fori_loop(0, PAIRS, pair2, 0)

    @pl.when(sid < NTAIL)
    def _():
        c = sid + NFULL * NS
        pltpu.sync_copy(packed_hbm.at[c, 1], dst0)
        pltpu.async_copy(ea_hbm.at[pl.ds(c * CH, CH)], ea0, si0).wait()

        def cp(r, _):
            rows0[r, pl.ds(0, DE)] = ea0[r, pl.ds(0, DE)]
            return 0
        lax.fori_loop(0, CH, cp, 0)
        compute_dloc(dst0, dloc0)
        pltpu.sync_copy(rows0, acc_sh.at[dloc0], add=True)

    plsc.subcore_barrier()
    writeout(out_ae)


@jax.jit
def _sc_edge_agg(packed, x, ea):
    mesh = plsc.VectorSubcoreMesh(core_axis_name="c", subcore_axis_name="s")
    fn = pl.kernel(
        _sc_edge_agg_body,
        out_type=[
            jax.ShapeDtypeStruct((N, D), jnp.float32),
            jax.ShapeDtypeStruct((N, D), jnp.float32),
        ],
        mesh=mesh,
        scratch_types=[
            pltpu.VMEM((CH,), jnp.int32),
            pltpu.VMEM((CH,), jnp.int32),
            pltpu.VMEM((CH,), jnp.int32),
            pltpu.VMEM((CH,), jnp.int32),
            pltpu.VMEM((CH,), jnp.int32),
            pltpu.VMEM((CH,), jnp.int32),
            pltpu.VMEM((CH, D), jnp.float32),
            pltpu.VMEM((CH, D), jnp.float32),
            pltpu.VMEM((CH, DE), jnp.float32),
            pltpu.VMEM((CH, DE), jnp.float32),
            pltpu.VMEM((ZR, D), jnp.float32),
            pltpu.VMEM((WR, D), jnp.float32),
            pltpu.VMEM_SHARED((NHP, D), jnp.float32),
            pltpu.SemaphoreType.DMA,
            pltpu.SemaphoreType.DMA,
            pltpu.SemaphoreType.DMA,
            pltpu.SemaphoreType.DMA,
            pltpu.SemaphoreType.DMA,
            pltpu.SemaphoreType.DMA,
        ],
    )
    return fn(packed, x, ea)


TN = 1000
GRID = N // TN


def _tc_dense_body(x_ref, ax_ref, ae_ref, bf_ref,
                   Wself_ref, Wnbr_ref, Wedge_ref, benc_ref,
                   We1_ref, be1_ref, We2_ref, be2_ref,
                   Wc1_ref, bc1_ref, Wc2_ref, bc2_ref,
                   Wv1_ref, bv1_ref, Wv2_ref, bv2_ref,
                   Whc_ref, bhc_ref, Whe_ref, bhe_ref,
                   mask_ref, predc_ref, prede_ref,
                   Gc, Gh, Cn):
    i = pl.program_id(0)

    @pl.when(i == 0)
    def _():
        Gc[...] = jnp.zeros_like(Gc)
        Gh[...] = jnp.zeros_like(Gh)
        Cn[...] = jnp.zeros_like(Cn)

    h = (x_ref[...] @ Wself_ref[...] + ax_ref[...] @ Wnbr_ref[...]
         + ae_ref[...] @ Wedge_ref[...])
    h = jnp.maximum(h + benc_ref[...], 0.0)

    m1 = jnp.maximum(h @ We1_ref[...] + be1_ref[...], 0.0)
    logits = m1 @ We2_ref[...] + be2_ref[0, 0]
    mask = 1.0 / (1.0 + jnp.exp(-logits))
    mask_ref[...] = mask
    hc = h * mask

    bf = bf_ref[...]
    cols = lax.broadcasted_iota(jnp.int32, (TN, B), 1).astype(jnp.float32)
    P = jnp.where(bf == cols, 1.0, 0.0)

    dims = (((0,), (0,)), ((), ()))
    Gc[...] += lax.dot_general(P, hc, dims, preferred_element_type=jnp.float32)
    Gh[...] += lax.dot_general(P, h, dims, preferred_element_type=jnp.float32)
    Cn[...] += lax.dot_general(P, jnp.ones((TN, D), jnp.float32), dims,
                               preferred_element_type=jnp.float32)

    @pl.when(i == GRID - 1)
    def _():
        cnt = jnp.maximum(Cn[...], 1.0)
        hgc = Gc[...] / cnt
        hge = (Gh[...] - Gc[...]) / cnt
        t = jnp.maximum(hgc @ Wc1_ref[...] + bc1_ref[...], 0.0)
        hsc = jnp.maximum(t @ Wc2_ref[...] + bc2_ref[...], 0.0)
        t = jnp.maximum(hge @ Wv1_ref[...] + bv1_ref[...], 0.0)
        hse = jnp.maximum(t @ Wv2_ref[...] + bv2_ref[...], 0.0)
        predc_ref[...] = hsc @ Whc_ref[...] + bhc_ref[...]
        prede_ref[...] = hse @ Whe_ref[...] + bhe_ref[...]


def _full(shape):
    return pl.BlockSpec(shape, lambda i: (0,) * len(shape))


@jax.jit
def _tc_dense(x, ax, ae, batch_f,
              W_self, W_nbr, W_edge, b_enc,
              We1, be1, We2, be2,
              Wc1, bc1, Wc2, bc2,
              Wv1, bv1, Wv2, bv2,
              Whc_p, bhc_p, Whe_p, bhe_p):
    return pl.pallas_call(
        _tc_dense_body,
        grid=(GRID,),
        in_specs=[
            pl.BlockSpec((TN, D), lambda i: (i, 0)),
            pl.BlockSpec((TN, D), lambda i: (i, 0)),
            pl.BlockSpec((TN, D), lambda i: (i, 0)),
            pl.BlockSpec((TN, 1), lambda i: (i, 0)),
            _full((D, D)), _full((D, D)), _full((D, D)), _full((1, D)),
            _full((D, D // 2)), _full((1, D // 2)), _full((D // 2, 1)),
            _full((1, 1)),
            _full((D, D)), _full((1, D)), _full((D, BOT)), _full((1, BOT)),
            _full((D, D)), _full((1, D)), _full((D, BOT)), _full((1, BOT)),
            _full((BOT, D)), _full((1, D)), _full((BOT, D)), _full((1, D)),
        ],
        out_specs=[
            pl.BlockSpec((TN, 1), lambda i: (i, 0)),
            pl.BlockSpec((B, D), lambda i: (0, 0)),
            pl.BlockSpec((B, D), lambda i: (0, 0)),
        ],
        out_shape=[
            jax.ShapeDtypeStruct((N, 1), jnp.float32),
            jax.ShapeDtypeStruct((B, D), jnp.float32),
            jax.ShapeDtypeStruct((B, D), jnp.float32),
        ],
        scratch_shapes=[
            pltpu.VMEM((B, D), jnp.float32),
            pltpu.VMEM((B, D), jnp.float32),
            pltpu.VMEM((B, D), jnp.float32),
        ],
    )(x, ax, ae, batch_f,
      W_self, W_nbr, W_edge, b_enc,
      We1, be1, We2, be2,
      Wc1, bc1, Wc2, bc2,
      Wv1, bv1, Wv2, bv2,
      Whc_p, bhc_p, Whe_p, bhe_p)


def kernel(x, edge_index, edge_attr, batch,
           W_self, W_nbr, W_edge, b_enc,
           We1, be1, We2, be2,
           Wc1, bc1, Wc2, bc2,
           Wv1, bv1, Wv2, bv2,
           Whc, bhc, Whe, bhe):
    src = edge_index[0].astype(jnp.int32)
    dst = edge_index[1].astype(jnp.int32)
    packed = jnp.stack(
        [src.reshape(NCHUNKS, CH), dst.reshape(NCHUNKS, CH)], axis=1)

    ax, ae = _sc_edge_agg(packed, x, edge_attr)
    W_edge_p = jnp.pad(W_edge, ((0, D - DE), (0, 0)))

    batch_f = batch.astype(jnp.float32).reshape(N, 1)
    pad = lambda w: jnp.pad(w, ((0, 0), (0, D - w.shape[1])))
    Whc_p = pad(Whc)
    Whe_p = pad(Whe)
    bhc_p = jnp.pad(bhc, (0, D - ND)).reshape(1, D)
    bhe_p = jnp.pad(bhe, (0, D - ND)).reshape(1, D)

    mask, predc_full, prede_full = _tc_dense(
        x, ax, ae, batch_f,
        W_self, W_nbr, W_edge_p, b_enc.reshape(1, D),
        We1, be1.reshape(1, D // 2), We2, be2.reshape(1, 1),
        Wc1, bc1.reshape(1, D), Wc2, bc2.reshape(1, BOT),
        Wv1, bv1.reshape(1, D), Wv2, bv2.reshape(1, BOT),
        Whc_p, bhc_p, Whe_p, bhe_p)

    return (predc_full[:, :ND], prede_full[:, :ND], mask)


# speedup vs baseline: 3.4700x; 1.0061x over previous
import functools

import jax
import jax.numpy as jnp
from jax import lax
from jax.experimental import pallas as pl
from jax.experimental.pallas import tpu as pltpu
from jax.experimental.pallas import tpu_sc as plsc

N = 10000
E = 320000
D = 128
DE = 16
B = 512
BOT = 256
ND = 5

NC = 2
NS = 16

CH = 128
NCHUNKS = E // CH
NFULL = NCHUNKS // NS
PAIRS = NFULL // 2
NTAIL = NCHUNKS - NFULL * NS

NH = N // NC
NHP = 5040
ZR = 80
NQZ = NHP // ZR
QIZ = (NQZ + NS - 1) // NS
WR = 40
NQW = NH // WR
QIW = (NQW + NS - 1) // NS


def _sc_edge_agg_body(packed_hbm, x_hbm, ea_hbm, out_ax, out_ae,
                      src0, src1, dst0, dst1, dloc0, dloc1,
                      rows0, rows1, ea0, ea1, zrow, bnc, acc_sh,
                      sg0, sg1, ss0, ss1, si0, si1):
    cid = lax.axis_index("c")
    sid = lax.axis_index("s")
    lo = cid * NH

    SRC = (src0, src1)
    DST = (dst0, dst1)
    DLOC = (dloc0, dloc1)
    ROWS = (rows0, rows1)
    EA = (ea0, ea1)
    SG = (sg0, sg0)
    SS = (ss0, ss0)
    SI = (si0, si0)

    def zero_zrow(i, _):
        for j in range(D // 16):
            zrow[i, pl.ds(j * 16, 16)] = jnp.zeros((16,), jnp.float32)
        return 0
    lax.fori_loop(0, ZR, zero_zrow, 0)

    def zero_acc(k, _):
        q = sid + k * NS

        @pl.when(q < NQZ)
        def _():
            pltpu.sync_copy(zrow, acc_sh.at[pl.ds(q * ZR, ZR)])
        return 0

    def fill_dummy(dloc):
        for j in range(CH // 16):
            dloc[pl.ds(j * 16, 16)] = jnp.full((16,), NH, jnp.int32)

    def compute_dloc(dst, dloc):
        for j in range(CH // 16):
            d = dst[pl.ds(j * 16, 16)]
            m = (d >= lo) & (d < lo + NH)
            dloc[pl.ds(j * 16, 16)] = jnp.where(m, d - lo, NH)

    def writeout(out_hbm):
        def wo(k, _):
            q = sid + k * NS

            @pl.when(q < NQW)
            def _():
                r = q * WR
                pltpu.sync_copy(acc_sh.at[pl.ds(r, WR)], bnc)
                pltpu.sync_copy(bnc, out_hbm.at[pl.ds(lo + r, WR)])
            return 0
        lax.fori_loop(0, QIW, wo, 0)

    lax.fori_loop(0, QIZ, zero_acc, 0)
    fill_dummy(dloc0)
    fill_dummy(dloc1)
    plsc.subcore_barrier()

    pltpu.async_copy(rows0, acc_sh.at[dloc0], ss0, add=True)
    pltpu.sync_copy(packed_hbm.at[sid, 0], src0)
    pltpu.sync_copy(packed_hbm.at[sid, 1], dst0)
    pltpu.async_copy(packed_hbm.at[sid + NS, 0], src1, si0)
    pltpu.async_copy(packed_hbm.at[sid + NS, 1], dst1, si0)
    pltpu.async_copy(x_hbm.at[src0], rows0, sg0)

    def pair1(k, _):
        i0 = 2 * k
        for b in (0, 1):
            nb = 1 - b
            ii = i0 + b

            @pl.when(ii + 1 < NFULL)
            def _():
                pltpu.make_async_copy(packed_hbm.at[0, 0], SRC[nb], SI[0]).wait()
                pltpu.make_async_copy(packed_hbm.at[0, 1], DST[nb], SI[0]).wait()
                pltpu.make_async_copy(ROWS[nb], acc_sh.at[DLOC[nb]], SS[0]).wait()
                pltpu.async_copy(x_hbm.at[SRC[nb]], ROWS[nb], SG[0])
            pltpu.make_async_copy(x_hbm.at[SRC[b]], ROWS[b], SG[0]).wait()
            compute_dloc(DST[b], DLOC[b])
            pltpu.async_copy(ROWS[b], acc_sh.at[DLOC[b]], SS[0], add=True)

            @pl.when(ii + 2 < NFULL)
            def _():
                c2 = sid + (ii + 2) * NS
                pltpu.async_copy(packed_hbm.at[c2, 0], SRC[b], SI[0])
                pltpu.async_copy(packed_hbm.at[c2, 1], DST[b], SI[0])
        return 0
    lax.fori_loop(0, PAIRS, pair1, 0)

    pltpu.make_async_copy(rows0, acc_sh.at[dloc0], ss0).wait()
    pltpu.make_async_copy(rows1, acc_sh.at[dloc1], ss0).wait()

    @pl.when(sid < NTAIL)
    def _():
        c = sid + NFULL * NS
        pltpu.sync_copy(packed_hbm.at[c, 0], src0)
        pltpu.sync_copy(packed_hbm.at[c, 1], dst0)
        pltpu.async_copy(x_hbm.at[src0], rows0, sg0).wait()
        compute_dloc(dst0, dloc0)
        pltpu.sync_copy(rows0, acc_sh.at[dloc0], add=True)

    plsc.subcore_barrier()
    writeout(out_ax)
    plsc.subcore_barrier()

    lax.fori_loop(0, QIZ, zero_acc, 0)

    def zero_rows(i, _):
        for j in range(D // 16):
            rows0[i, pl.ds(j * 16, 16)] = jnp.zeros((16,), jnp.float32)
            rows1[i, pl.ds(j * 16, 16)] = jnp.zeros((16,), jnp.float32)
        return 0
    lax.fori_loop(0, CH, zero_rows, 0)
    fill_dummy(dloc0)
    fill_dummy(dloc1)
    plsc.subcore_barrier()

    pltpu.async_copy(packed_hbm.at[sid, 1], dst0, si0)
    pltpu.async_copy(ea_hbm.at[pl.ds(sid * CH, CH)], ea0, si0)
    pltpu.async_copy(packed_hbm.at[sid + NS, 1], dst1, si0)
    pltpu.async_copy(ea_hbm.at[pl.ds((sid + NS) * CH, CH)], ea1, si0)

    def pair2(k, _):
        i0 = 2 * k
        for b in (0, 1):
            ii = i0 + b
            pltpu.make_async_copy(packed_hbm.at[0, 1], DST[b], SI[b]).wait()
            pltpu.make_async_copy(ea_hbm.at[pl.ds(0, CH)], EA[b], SI[b]).wait()

            def cp(r, _):
                ROWS[b][r, pl.ds(0, DE)] = EA[b][r, pl.ds(0, DE)]
                return 0
            lax.fori_loop(0, CH, cp, 0)
            compute_dloc(DST[b], DLOC[b])

            @pl.when(ii + 2 < NFULL)
            def _():
                c2 = sid + (ii + 2) * NS
                pltpu.async_copy(packed_hbm.at[c2, 1], DST[b], SI[b])
                pltpu.async_copy(ea_hbm.at[pl.ds(c2 * CH, CH)], EA[b], SI[b])
            pltpu.sync_copy(ROWS[b], acc_sh.at[DLOC[b]], add=True)
        return 0
    lax.fori_loop(0, PAIRS, pair2, 0)

    @pl.when(sid < NTAIL)
    def _():
        c = sid + NFULL * NS
        pltpu.sync_copy(packed_hbm.at[c, 1], dst0)
        pltpu.async_copy(ea_hbm.at[pl.ds(c * CH, CH)], ea0, si0).wait()

        def cp(r, _):
            rows0[r, pl.ds(0, DE)] = ea0[r, pl.ds(0, DE)]
            return 0
        lax.fori_loop(0, CH, cp, 0)
        compute_dloc(dst0, dloc0)
        pltpu.sync_copy(rows0, acc_sh.at[dloc0], add=True)

    plsc.subcore_barrier()
    writeout(out_ae)


@jax.jit
def _sc_edge_agg(packed, x, ea):
    mesh = plsc.VectorSubcoreMesh(core_axis_name="c", subcore_axis_name="s")
    fn = pl.kernel(
        _sc_edge_agg_body,
        out_type=[
            jax.ShapeDtypeStruct((N, D), jnp.float32),
            jax.ShapeDtypeStruct((N, D), jnp.float32),
        ],
        mesh=mesh,
        scratch_types=[
            pltpu.VMEM((CH,), jnp.int32),
            pltpu.VMEM((CH,), jnp.int32),
            pltpu.VMEM((CH,), jnp.int32),
            pltpu.VMEM((CH,), jnp.int32),
            pltpu.VMEM((CH,), jnp.int32),
            pltpu.VMEM((CH,), jnp.int32),
            pltpu.VMEM((CH, D), jnp.float32),
            pltpu.VMEM((CH, D), jnp.float32),
            pltpu.VMEM((CH, DE), jnp.float32),
            pltpu.VMEM((CH, DE), jnp.float32),
            pltpu.VMEM((ZR, D), jnp.float32),
            pltpu.VMEM((WR, D), jnp.float32),
            pltpu.VMEM_SHARED((NHP, D), jnp.float32),
            pltpu.SemaphoreType.DMA,
            pltpu.SemaphoreType.DMA,
            pltpu.SemaphoreType.DMA,
            pltpu.SemaphoreType.DMA,
            pltpu.SemaphoreType.DMA,
            pltpu.SemaphoreType.DMA,
        ],
    )
    return fn(packed, x, ea)


TN = 1000
GRID = N // TN


def _tc_dense_body(x_ref, ax_ref, ae_ref, bf_ref,
                   Wself_ref, Wnbr_ref, Wedge_ref, benc_ref,
                   We1_ref, be1_ref, We2_ref, be2_ref,
                   Wc1_ref, bc1_ref, Wc2_ref, bc2_ref,
                   Wv1_ref, bv1_ref, Wv2_ref, bv2_ref,
                   Whc_ref, bhc_ref, Whe_ref, bhe_ref,
                   mask_ref, predc_ref, prede_ref,
                   Gc, Gh, Cn):
    i = pl.program_id(0)

    @pl.when(i == 0)
    def _():
        Gc[...] = jnp.zeros_like(Gc)
        Gh[...] = jnp.zeros_like(Gh)
        Cn[...] = jnp.zeros_like(Cn)

    h = (x_ref[...] @ Wself_ref[...] + ax_ref[...] @ Wnbr_ref[...]
         + ae_ref[...] @ Wedge_ref[...])
    h = jnp.maximum(h + benc_ref[...], 0.0)

    m1 = jnp.maximum(h @ We1_ref[...] + be1_ref[...], 0.0)
    logits = m1 @ We2_ref[...] + be2_ref[0, 0]
    mask = 1.0 / (1.0 + jnp.exp(-logits))
    mask_ref[...] = mask
    hc = h * mask

    bf = bf_ref[...]
    cols = lax.broadcasted_iota(jnp.int32, (TN, B), 1).astype(jnp.float32)
    P = jnp.where(bf == cols, 1.0, 0.0)

    dims = (((0,), (0,)), ((), ()))
    Gc[...] += lax.dot_general(P, hc, dims, preferred_element_type=jnp.float32)
    Gh[...] += lax.dot_general(P, h, dims, preferred_element_type=jnp.float32)
    Cn[...] += lax.dot_general(P, jnp.ones((TN, D), jnp.float32), dims,
                               preferred_element_type=jnp.float32)

    @pl.when(i == GRID - 1)
    def _():
        cnt = jnp.maximum(Cn[...], 1.0)
        hgc = Gc[...] / cnt
        hge = (Gh[...] - Gc[...]) / cnt
        t = jnp.maximum(hgc @ Wc1_ref[...] + bc1_ref[...], 0.0)
        hsc = jnp.maximum(t @ Wc2_ref[...] + bc2_ref[...], 0.0)
        t = jnp.maximum(hge @ Wv1_ref[...] + bv1_ref[...], 0.0)
        hse = jnp.maximum(t @ Wv2_ref[...] + bv2_ref[...], 0.0)
        predc_ref[...] = hsc @ Whc_ref[...] + bhc_ref[...]
        prede_ref[...] = hse @ Whe_ref[...] + bhe_ref[...]


def _full(shape):
    return pl.BlockSpec(shape, lambda i: (0,) * len(shape))


@jax.jit
def _tc_dense(x, ax, ae, batch_f,
              W_self, W_nbr, W_edge, b_enc,
              We1, be1, We2, be2,
              Wc1, bc1, Wc2, bc2,
              Wv1, bv1, Wv2, bv2,
              Whc_p, bhc_p, Whe_p, bhe_p):
    return pl.pallas_call(
        _tc_dense_body,
        grid=(GRID,),
        in_specs=[
            pl.BlockSpec((TN, D), lambda i: (i, 0)),
            pl.BlockSpec((TN, D), lambda i: (i, 0)),
            pl.BlockSpec((TN, D), lambda i: (i, 0)),
            pl.BlockSpec((TN, 1), lambda i: (i, 0)),
            _full((D, D)), _full((D, D)), _full((D, D)), _full((1, D)),
            _full((D, D // 2)), _full((1, D // 2)), _full((D // 2, 1)),
            _full((1, 1)),
            _full((D, D)), _full((1, D)), _full((D, BOT)), _full((1, BOT)),
            _full((D, D)), _full((1, D)), _full((D, BOT)), _full((1, BOT)),
            _full((BOT, D)), _full((1, D)), _full((BOT, D)), _full((1, D)),
        ],
        out_specs=[
            pl.BlockSpec((TN, 1), lambda i: (i, 0)),
            pl.BlockSpec((B, D), lambda i: (0, 0)),
            pl.BlockSpec((B, D), lambda i: (0, 0)),
        ],
        out_shape=[
            jax.ShapeDtypeStruct((N, 1), jnp.float32),
            jax.ShapeDtypeStruct((B, D), jnp.float32),
            jax.ShapeDtypeStruct((B, D), jnp.float32),
        ],
        scratch_shapes=[
            pltpu.VMEM((B, D), jnp.float32),
            pltpu.VMEM((B, D), jnp.float32),
            pltpu.VMEM((B, D), jnp.float32),
        ],
    )(x, ax, ae, batch_f,
      W_self, W_nbr, W_edge, b_enc,
      We1, be1, We2, be2,
      Wc1, bc1, Wc2, bc2,
      Wv1, bv1, Wv2, bv2,
      Whc_p, bhc_p, Whe_p, bhe_p)


def kernel(x, edge_index, edge_attr, batch,
           W_self, W_nbr, W_edge, b_enc,
           We1, be1, We2, be2,
           Wc1, bc1, Wc2, bc2,
           Wv1, bv1, Wv2, bv2,
           Whc, bhc, Whe, bhe):
    src = edge_index[0].astype(jnp.int32)
    dst = edge_index[1].astype(jnp.int32)
    packed = jnp.stack(
        [src.reshape(NCHUNKS, CH), dst.reshape(NCHUNKS, CH)], axis=1)

    ax, ae = _sc_edge_agg(packed, x, edge_attr)
    W_edge_p = jnp.pad(W_edge, ((0, D - DE), (0, 0)))

    batch_f = batch.astype(jnp.float32).reshape(N, 1)
    pad = lambda w: jnp.pad(w, ((0, 0), (0, D - w.shape[1])))
    Whc_p = pad(Whc)
    Whe_p = pad(Whe)
    bhc_p = jnp.pad(bhc, (0, D - ND)).reshape(1, D)
    bhe_p = jnp.pad(bhe, (0, D - ND)).reshape(1, D)

    mask, predc_full, prede_full = _tc_dense(
        x, ax, ae, batch_f,
        W_self, W_nbr, W_edge_p, b_enc.reshape(1, D),
        We1, be1.reshape(1, D // 2), We2, be2.reshape(1, 1),
        Wc1, bc1.reshape(1, D), Wc2, bc2.reshape(1, BOT),
        Wv1, bv1.reshape(1, D), Wv2, bv2.reshape(1, BOT),
        Whc_p, bhc_p, Whe_p, bhe_p)

    return (predc_full[:, :ND], prede_full[:, :ND], mask)


# speedup vs baseline: 3.4711x; 1.0003x over previous
import jax
import jax.numpy as jnp
from jax import lax
from jax.experimental import pallas as pl
from jax.experimental.pallas import tpu as pltpu
from jax.experimental.pallas import tpu_sc as plsc

N = 10000
E = 320000
D = 128
DE = 16
B = 512
BOT = 256
ND = 5

NC = 2
NS = 16

CH = 128
NCHUNKS = E // CH
NFULL = NCHUNKS // NS
PAIRS = NFULL // 2
NTAIL = NCHUNKS - NFULL * NS

NH = N // NC
NHP = 5040
ZR = 80
NQZ = NHP // ZR
QIZ = (NQZ + NS - 1) // NS
WR = 40
NQW = NH // WR
QIW = (NQW + NS - 1) // NS


def _sc_edge_agg_body(packed_hbm, x_hbm, ea_hbm, out_ax, out_ae,
                      src0, src1, dst0, dst1, dloc0, dloc1,
                      rows0, rows1, ea0, ea1, zrow, bnc, acc_sh,
                      sg0, sg1, ss0, ss1, si0, si1):
    cid = lax.axis_index("c")
    sid = lax.axis_index("s")
    lo = cid * NH

    SRC = (src0, src1)
    DST = (dst0, dst1)
    DLOC = (dloc0, dloc1)
    ROWS = (rows0, rows1)
    EA = (ea0, ea1)
    SG = (sg0, sg0)
    SS = (ss0, ss0)
    SI = (si0, si0)

    def zero_zrow(i, _):
        for j in range(D // 16):
            zrow[i, pl.ds(j * 16, 16)] = jnp.zeros((16,), jnp.float32)
        return 0
    lax.fori_loop(0, ZR, zero_zrow, 0)

    def zero_acc(k, _):
        q = sid + k * NS

        @pl.when(q < NQZ)
        def _():
            pltpu.sync_copy(zrow, acc_sh.at[pl.ds(q * ZR, ZR)])
        return 0

    def fill_dummy(dloc):
        for j in range(CH // 16):
            dloc[pl.ds(j * 16, 16)] = jnp.full((16,), NH, jnp.int32)

    def compute_dloc(dst, dloc):
        for j in range(CH // 16):
            d = dst[pl.ds(j * 16, 16)]
            m = (d >= lo) & (d < lo + NH)
            dloc[pl.ds(j * 16, 16)] = jnp.where(m, d - lo, NH)

    def writeout(out_hbm):
        def wo(k, _):
            q = sid + k * NS

            @pl.when(q < NQW)
            def _():
                r = q * WR
                pltpu.sync_copy(acc_sh.at[pl.ds(r, WR)], bnc)
                pltpu.sync_copy(bnc, out_hbm.at[pl.ds(lo + r, WR)])
            return 0
        lax.fori_loop(0, QIW, wo, 0)

    lax.fori_loop(0, QIZ, zero_acc, 0)
    fill_dummy(dloc0)
    fill_dummy(dloc1)
    plsc.subcore_barrier()

    pltpu.async_copy(rows0, acc_sh.at[dloc0], ss0, add=True)
    pltpu.sync_copy(packed_hbm.at[sid, 0], src0)
    pltpu.sync_copy(packed_hbm.at[sid, 1], dst0)
    pltpu.async_copy(packed_hbm.at[sid + NS, 0], src1, si0)
    pltpu.async_copy(packed_hbm.at[sid + NS, 1], dst1, si0)
    pltpu.async_copy(x_hbm.at[src0], rows0, sg0)

    def pair1(k, _):
        i0 = 2 * k
        for b in (0, 1):
            nb = 1 - b
            ii = i0 + b

            @pl.when(ii + 1 < NFULL)
            def _():
                pltpu.make_async_copy(packed_hbm.at[0, 0], SRC[nb], SI[0]).wait()
                pltpu.make_async_copy(packed_hbm.at[0, 1], DST[nb], SI[0]).wait()
                pltpu.make_async_copy(ROWS[nb], acc_sh.at[DLOC[nb]], SS[0]).wait()
                pltpu.async_copy(x_hbm.at[SRC[nb]], ROWS[nb], SG[0])
            pltpu.make_async_copy(x_hbm.at[SRC[b]], ROWS[b], SG[0]).wait()
            compute_dloc(DST[b], DLOC[b])
            pltpu.async_copy(ROWS[b], acc_sh.at[DLOC[b]], SS[0], add=True)

            @pl.when(ii + 2 < NFULL)
            def _():
                c2 = sid + (ii + 2) * NS
                pltpu.async_copy(packed_hbm.at[c2, 0], SRC[b], SI[0])
                pltpu.async_copy(packed_hbm.at[c2, 1], DST[b], SI[0])
        return 0
    lax.fori_loop(0, PAIRS, pair1, 0)

    pltpu.make_async_copy(rows0, acc_sh.at[dloc0], ss0).wait()
    pltpu.make_async_copy(rows1, acc_sh.at[dloc1], ss0).wait()

    @pl.when(sid < NTAIL)
    def _():
        c = sid + NFULL * NS
        pltpu.sync_copy(packed_hbm.at[c, 0], src0)
        pltpu.sync_copy(packed_hbm.at[c, 1], dst0)
        pltpu.async_copy(x_hbm.at[src0], rows0, sg0).wait()
        compute_dloc(dst0, dloc0)
        pltpu.sync_copy(rows0, acc_sh.at[dloc0], add=True)

    plsc.subcore_barrier()
    writeout(out_ax)
    plsc.subcore_barrier()

    lax.fori_loop(0, QIZ, zero_acc, 0)

    def zero_rows(i, _):
        for j in range(D // 16):
            rows0[i, pl.ds(j * 16, 16)] = jnp.zeros((16,), jnp.float32)
            rows1[i, pl.ds(j * 16, 16)] = jnp.zeros((16,), jnp.float32)
        return 0
    lax.fori_loop(0, CH, zero_rows, 0)
    fill_dummy(dloc0)
    fill_dummy(dloc1)
    plsc.subcore_barrier()

    pltpu.async_copy(packed_hbm.at[sid, 1], dst0, si0)
    pltpu.async_copy(ea_hbm.at[pl.ds(sid * CH, CH)], ea0, si0)
    pltpu.async_copy(packed_hbm.at[sid + NS, 1], dst1, si0)
    pltpu.async_copy(ea_hbm.at[pl.ds((sid + NS) * CH, CH)], ea1, si0)

    def pair2(k, _):
        i0 = 2 * k
        for b in (0, 1):
            ii = i0 + b
            pltpu.make_async_copy(packed_hbm.at[0, 1], DST[b], SI[b]).wait()
            pltpu.make_async_copy(ea_hbm.at[pl.ds(0, CH)], EA[b], SI[b]).wait()

            def cp(r, _):
                ROWS[b][r, pl.ds(0, DE)] = EA[b][r, pl.ds(0, DE)]
                return 0
            lax.fori_loop(0, CH, cp, 0)
            compute_dloc(DST[b], DLOC[b])

            @pl.when(ii + 2 < NFULL)
            def _():
                c2 = sid + (ii + 2) * NS
                pltpu.async_copy(packed_hbm.at[c2, 1], DST[b], SI[b])
                pltpu.async_copy(ea_hbm.at[pl.ds(c2 * CH, CH)], EA[b], SI[b])
            pltpu.sync_copy(ROWS[b], acc_sh.at[DLOC[b]], add=True)
        return 0
    lax.fori_loop(0, PAIRS, pair2, 0)

    @pl.when(sid < NTAIL)
    def _():
        c = sid + NFULL * NS
        pltpu.sync_copy(packed_hbm.at[c, 1], dst0)
        pltpu.async_copy(ea_hbm.at[pl.ds(c * CH, CH)], ea0, si0).wait()

        def cp(r, _):
            rows0[r, pl.ds(0, DE)] = ea0[r, pl.ds(0, DE)]
            return 0
        lax.fori_loop(0, CH, cp, 0)
        compute_dloc(dst0, dloc0)
        pltpu.sync_copy(rows0, acc_sh.at[dloc0], add=True)

    plsc.subcore_barrier()
    writeout(out_ae)


@jax.jit
def _sc_edge_agg(packed, x, ea):
    mesh = plsc.VectorSubcoreMesh(core_axis_name="c", subcore_axis_name="s")
    fn = pl.kernel(
        _sc_edge_agg_body,
        out_type=[
            jax.ShapeDtypeStruct((N, D), jnp.float32),
            jax.ShapeDtypeStruct((N, D), jnp.float32),
        ],
        mesh=mesh,
        scratch_types=[
            pltpu.VMEM((CH,), jnp.int32),
            pltpu.VMEM((CH,), jnp.int32),
            pltpu.VMEM((CH,), jnp.int32),
            pltpu.VMEM((CH,), jnp.int32),
            pltpu.VMEM((CH,), jnp.int32),
            pltpu.VMEM((CH,), jnp.int32),
            pltpu.VMEM((CH, D), jnp.float32),
            pltpu.VMEM((CH, D), jnp.float32),
            pltpu.VMEM((CH, DE), jnp.float32),
            pltpu.VMEM((CH, DE), jnp.float32),
            pltpu.VMEM((ZR, D), jnp.float32),
            pltpu.VMEM((WR, D), jnp.float32),
            pltpu.VMEM_SHARED((NHP, D), jnp.float32),
            pltpu.SemaphoreType.DMA,
            pltpu.SemaphoreType.DMA,
            pltpu.SemaphoreType.DMA,
            pltpu.SemaphoreType.DMA,
            pltpu.SemaphoreType.DMA,
            pltpu.SemaphoreType.DMA,
        ],
    )
    return fn(packed, x, ea)


TN = 1000
GRID = N // TN


def _tc_dense_body(x_ref, ax_ref, ae_ref, bf_ref,
                   Wself_ref, Wnbr_ref, Wedge_ref, benc_ref,
                   We1_ref, be1_ref, We2_ref, be2_ref,
                   Wc1_ref, bc1_ref, Wc2_ref, bc2_ref,
                   Wv1_ref, bv1_ref, Wv2_ref, bv2_ref,
                   Whc_ref, bhc_ref, Whe_ref, bhe_ref,
                   mask_ref, predc_ref, prede_ref,
                   Gc, Gh, Cn):
    i = pl.program_id(0)

    @pl.when(i == 0)
    def _():
        Gc[...] = jnp.zeros_like(Gc)
        Gh[...] = jnp.zeros_like(Gh)
        Cn[...] = jnp.zeros_like(Cn)

    h = (x_ref[...] @ Wself_ref[...] + ax_ref[...] @ Wnbr_ref[...]
         + ae_ref[...] @ Wedge_ref[...])
    h = jnp.maximum(h + benc_ref[...], 0.0)

    m1 = jnp.maximum(h @ We1_ref[...] + be1_ref[...], 0.0)
    logits = m1 @ We2_ref[...] + be2_ref[0, 0]
    mask = 1.0 / (1.0 + jnp.exp(-logits))
    mask_ref[...] = mask
    hc = h * mask

    bf = bf_ref[...]
    cols = lax.broadcasted_iota(jnp.int32, (TN, B), 1).astype(jnp.float32)
    P = jnp.where(bf == cols, 1.0, 0.0)

    dims = (((0,), (0,)), ((), ()))
    Gc[...] += lax.dot_general(P, hc, dims, preferred_element_type=jnp.float32)
    Gh[...] += lax.dot_general(P, h, dims, preferred_element_type=jnp.float32)
    Cn[...] += lax.dot_general(P, jnp.ones((TN, D), jnp.float32), dims,
                               preferred_element_type=jnp.float32)

    @pl.when(i == GRID - 1)
    def _():
        cnt = jnp.maximum(Cn[...], 1.0)
        hgc = Gc[...] / cnt
        hge = (Gh[...] - Gc[...]) / cnt
        t = jnp.maximum(hgc @ Wc1_ref[...] + bc1_ref[...], 0.0)
        hsc = jnp.maximum(t @ Wc2_ref[...] + bc2_ref[...], 0.0)
        t = jnp.maximum(hge @ Wv1_ref[...] + bv1_ref[...], 0.0)
        hse = jnp.maximum(t @ Wv2_ref[...] + bv2_ref[...], 0.0)
        predc_ref[...] = hsc @ Whc_ref[...] + bhc_ref[...]
        prede_ref[...] = hse @ Whe_ref[...] + bhe_ref[...]


def _full(shape):
    return pl.BlockSpec(shape, lambda i: (0,) * len(shape))


@jax.jit
def _tc_dense(x, ax, ae, batch_f,
              W_self, W_nbr, W_edge, b_enc,
              We1, be1, We2, be2,
              Wc1, bc1, Wc2, bc2,
              Wv1, bv1, Wv2, bv2,
              Whc_p, bhc_p, Whe_p, bhe_p):
    return pl.pallas_call(
        _tc_dense_body,
        grid=(GRID,),
        in_specs=[
            pl.BlockSpec((TN, D), lambda i: (i, 0)),
            pl.BlockSpec((TN, D), lambda i: (i, 0)),
            pl.BlockSpec((TN, D), lambda i: (i, 0)),
            pl.BlockSpec((TN, 1), lambda i: (i, 0)),
            _full((D, D)), _full((D, D)), _full((D, D)), _full((1, D)),
            _full((D, D // 2)), _full((1, D // 2)), _full((D // 2, 1)),
            _full((1, 1)),
            _full((D, D)), _full((1, D)), _full((D, BOT)), _full((1, BOT)),
            _full((D, D)), _full((1, D)), _full((D, BOT)), _full((1, BOT)),
            _full((BOT, D)), _full((1, D)), _full((BOT, D)), _full((1, D)),
        ],
        out_specs=[
            pl.BlockSpec((TN, 1), lambda i: (i, 0)),
            pl.BlockSpec((B, D), lambda i: (0, 0)),
            pl.BlockSpec((B, D), lambda i: (0, 0)),
        ],
        out_shape=[
            jax.ShapeDtypeStruct((N, 1), jnp.float32),
            jax.ShapeDtypeStruct((B, D), jnp.float32),
            jax.ShapeDtypeStruct((B, D), jnp.float32),
        ],
        scratch_shapes=[
            pltpu.VMEM((B, D), jnp.float32),
            pltpu.VMEM((B, D), jnp.float32),
            pltpu.VMEM((B, D), jnp.float32),
        ],
    )(x, ax, ae, batch_f,
      W_self, W_nbr, W_edge, b_enc,
      We1, be1, We2, be2,
      Wc1, bc1, Wc2, bc2,
      Wv1, bv1, Wv2, bv2,
      Whc_p, bhc_p, Whe_p, bhe_p)


def kernel(x, edge_index, edge_attr, batch,
           W_self, W_nbr, W_edge, b_enc,
           We1, be1, We2, be2,
           Wc1, bc1, Wc2, bc2,
           Wv1, bv1, Wv2, bv2,
           Whc, bhc, Whe, bhe):
    src = edge_index[0].astype(jnp.int32)
    dst = edge_index[1].astype(jnp.int32)
    packed = jnp.stack(
        [src.reshape(NCHUNKS, CH), dst.reshape(NCHUNKS, CH)], axis=1)

    ax, ae = _sc_edge_agg(packed, x, edge_attr)
    W_edge_p = jnp.pad(W_edge, ((0, D - DE), (0, 0)))

    batch_f = batch.astype(jnp.float32).reshape(N, 1)
    pad = lambda w: jnp.pad(w, ((0, 0), (0, D - w.shape[1])))
    Whc_p = pad(Whc)
    Whe_p = pad(Whe)
    bhc_p = jnp.pad(bhc, (0, D - ND)).reshape(1, D)
    bhe_p = jnp.pad(bhe, (0, D - ND)).reshape(1, D)

    mask, predc_full, prede_full = _tc_dense(
        x, ax, ae, batch_f,
        W_self, W_nbr, W_edge_p, b_enc.reshape(1, D),
        We1, be1.reshape(1, D // 2), We2, be2.reshape(1, 1),
        Wc1, bc1.reshape(1, D), Wc2, bc2.reshape(1, BOT),
        Wv1, bv1.reshape(1, D), Wv2, bv2.reshape(1, BOT),
        Whc_p, bhc_p, Whe_p, bhe_p)

    return (predc_full[:, :ND], prede_full[:, :ND], mask)
